# Initial kernel scaffold; baseline (speedup 1.0000x reference)
import jax
import jax.numpy as jnp
from jax.experimental import pallas as pl


def kernel(x, edge_index, W1, b1, g1, bt1, W2, b2, g2, bt2):
    raise NotImplementedError("write your pallas kernel here")


# speedup vs baseline: 1.0240x; 1.0240x over previous
import functools
import jax
import jax.numpy as jnp
from jax.experimental import pallas as pl
from jax.experimental.pallas import tpu as pltpu

N_NODES_ = 10000
ROW_BLK = 1000


def _dense_layer_body(x_ref, a_ref, wa_ref, wb_ref, b_ref, g_ref, bt_ref, o_ref,
                      *, extra_relu):
    x = x_ref[...]
    a = a_ref[...]
    a = jnp.where(jnp.isfinite(a), a, 0.0)
    h = (jnp.dot(x, wa_ref[...], preferred_element_type=jnp.float32)
         + jnp.dot(a, wb_ref[...], preferred_element_type=jnp.float32)
         + b_ref[...])
    h = jnp.maximum(h, 0.0)
    mean = jnp.mean(h, axis=-1, keepdims=True)
    var = jnp.mean((h - mean) * (h - mean), axis=-1, keepdims=True)
    h = (h - mean) * jax.lax.rsqrt(var + 1e-5) * g_ref[...] + bt_ref[...]
    if extra_relu:
        h = jnp.maximum(h, 0.0)
    o_ref[...] = h


def _dense_layer(x, aggr, W, b, g, bt, d_out, extra_relu):
    d_in = x.shape[1]
    n = x.shape[0]
    wa = W[:d_in]
    wb = W[d_in:]
    grid = (n // ROW_BLK,)
    return pl.pallas_call(
        functools.partial(_dense_layer_body, extra_relu=extra_relu),
        grid=grid,
        in_specs=[
            pl.BlockSpec((ROW_BLK, d_in), lambda i: (i, 0)),
            pl.BlockSpec((ROW_BLK, d_in), lambda i: (i, 0)),
            pl.BlockSpec((d_in, d_out), lambda i: (0, 0)),
            pl.BlockSpec((d_in, d_out), lambda i: (0, 0)),
            pl.BlockSpec((d_out,), lambda i: (0,)),
            pl.BlockSpec((d_out,), lambda i: (0,)),
            pl.BlockSpec((d_out,), lambda i: (0,)),
        ],
        out_specs=pl.BlockSpec((ROW_BLK, d_out), lambda i: (i, 0)),
        out_shape=jax.ShapeDtypeStruct((n, d_out), jnp.float32),
    )(x, aggr, wa, wb, b, g, bt)


def kernel(x, edge_index, W1, b1, g1, bt1, W2, b2, g2, bt2):
    src = jnp.asarray(edge_index[0], jnp.int32)
    dst = jnp.asarray(edge_index[1], jnp.int32)
    n = x.shape[0]

    msgs = jnp.take(x, src, axis=0)
    aggr1 = jax.ops.segment_max(msgs, dst, num_segments=n)
    h = _dense_layer(x, aggr1, W1, b1, g1, bt1, 128, extra_relu=True)

    msgs2 = jnp.take(h, src, axis=0)
    aggr2 = jax.ops.segment_max(msgs2, dst, num_segments=n)
    out = _dense_layer(h, aggr2, W2, b2, g2, bt2, 64, extra_relu=False)
    return out


# speedup vs baseline: 3.1200x; 3.0470x over previous
import functools
import jax
import jax.numpy as jnp
from jax import lax
from jax.experimental import pallas as pl
from jax.experimental.pallas import tpu as pltpu
from jax.experimental.pallas import tpu_sc as plsc

N = 10000
E = 320000
D = 128
NW = 32
SEG = 320
NPAD = NW * SEG
CHUNK = 3200
NGROUP = CHUNK // 16
UNROLL = 5
CAP = 8192
SPILL = 512
EC = E + 64
SUP = 256
GB = 64
NB = SUP // GB

_mesh = plsc.VectorSubcoreMesh(core_axis_name="c", subcore_axis_name="s")


def _worker_id():
    return lax.axis_index("s") * 2 + lax.axis_index("c")


def _init_acc(acc):
    def body(r, _):
        for k in range(8):
            acc[r, pl.ds(k * 16, 16)] = jnp.full((16,), -jnp.inf, jnp.float32)
        return 0
    lax.fori_loop(0, SEG + 1, body, 0)


def _apply_phase(feat, clist, acc, bufs, wid, cnt):
    (pbufA, cidxA, clocA, rowsA, gsemA), (pbufB, cidxB, clocB, rowsB, gsemB) = bufs
    r64 = ((cnt + (GB - 1)) // GB) * GB
    nsup = (cnt + (SUP - 1)) // SUP

    def stage(s, pbuf, cidx, cloc, rows, gsem):
        base = s * SUP
        off = pl.multiple_of(wid * EC + base, 8)
        pltpu.sync_copy(clist.at[pl.ds(off, SUP)], pbuf)

        def unpack(g, _):
            v = pbuf[pl.ds(g * 16, 16)]
            cidx[pl.ds(g * 16, 16)] = v & 16383
            cloc[pl.ds(g * 16, 16)] = lax.shift_right_logical(v, 14)
            return 0
        lax.fori_loop(0, SUP // 16, unpack, 0)

        for b in range(NB):
            @pl.when(base + b * GB < cnt)
            def _():
                pltpu.async_copy(
                    feat.at[cidx.at[pl.ds(b * GB, GB)]],
                    rows.at[pl.ds(b * GB, GB)], gsem)

    def drain_apply(s, cidx, cloc, rows, gsem):
        base = s * SUP
        for b in range(NB):
            @pl.when(base + b * GB < cnt)
            def _():
                pltpu.make_async_copy(
                    feat.at[cidx.at[pl.ds(b * GB, GB)]],
                    rows.at[pl.ds(b * GB, GB)], gsem).wait()

        lim = jnp.minimum(SUP, r64 - base)

        def apply_grp(j, _):
            locv = cloc[pl.ds(j * 16, 16)]
            for t in range(16):
                lv = locv[t]
                i = j * 16 + t
                for k in range(8):
                    sl = pl.ds(k * 16, 16)
                    acc[lv, sl] = jnp.maximum(acc[lv, sl], rows[i, sl])
            return 0
        lax.fori_loop(0, lim // 16, apply_grp, 0)

    @pl.when(nsup > 0)
    def _():
        stage(0, pbufA, cidxA, clocA, rowsA, gsemA)

    def loop_body(s, _):
        nxt = s + 1
        even = s % 2 == 0

        @pl.when((nxt < nsup) & even)
        def _():
            stage(nxt, pbufB, cidxB, clocB, rowsB, gsemB)

        @pl.when((nxt < nsup) & (~even))
        def _():
            stage(nxt, pbufA, cidxA, clocA, rowsA, gsemA)

        @pl.when(even)
        def _():
            drain_apply(s, cidxA, clocA, rowsA, gsemA)

        @pl.when(~even)
        def _():
            drain_apply(s, cidxB, clocB, rowsB, gsemB)
        return 0

    lax.fori_loop(0, nsup, loop_body, 0)


def _sc_layer1_body(feat, srcl, dstl, aggr, clist, counts,
                    sbufA, dbufA, sbufB, dbufB, ring,
                    pbufA, cidxA, clocA, rowsA,
                    pbufB, cidxB, clocB, rowsB,
                    cntb, acc, esemA, esemB, gsemA, gsemB):
    wid = _worker_id()
    lo = pl.multiple_of(wid * SEG, 8)
    _init_acc(acc)

    nchunks = E // CHUNK

    def fetch(c, sbuf, dbuf, esem):
        pltpu.async_copy(srcl.at[pl.ds(c * CHUNK, CHUNK)], sbuf, esem)
        pltpu.async_copy(dstl.at[pl.ds(c * CHUNK, CHUNK)], dbuf, esem)

    def wait_fetch(c, sbuf, dbuf, esem):
        pltpu.make_async_copy(srcl.at[pl.ds(c * CHUNK, CHUNK)], sbuf, esem).wait()
        pltpu.make_async_copy(dstl.at[pl.ds(c * CHUNK, CHUNK)], dbuf, esem).wait()

    fetch(0, sbufA, dbufA, esemA)

    def filter_chunk(c, sbuf, dbuf, esem, tail):
        wait_fetch(c, sbuf, dbuf, esem)

        def grp_at(g, tl):
            sl = pl.ds(g * 16, 16)
            d16 = dbuf[sl]
            s16 = sbuf[sl]
            loc = d16 - lo
            m = (loc >= 0) & (loc < SEG)
            cs = plsc.cumsum(m.astype(jnp.int32))
            pos = (tl + cs - 1) & (CAP - 1)
            packed = s16 | lax.shift_left(loc, 14)
            plsc.store_scatter(ring, [pos], packed, mask=m)
            return tl + cs[15]

        def grp_blk(q, tl):
            for u in range(UNROLL):
                tl = grp_at(q * UNROLL + u, tl)
            return tl

        return lax.fori_loop(0, NGROUP // UNROLL, grp_blk, tail)

    def chunk_body(c, carry):
        tail, head = carry
        even = c % 2 == 0

        nxt = c + 1

        @pl.when((nxt < nchunks) & even)
        def _():
            fetch(nxt, sbufB, dbufB, esemB)

        @pl.when((nxt < nchunks) & (~even))
        def _():
            fetch(nxt, sbufA, dbufA, esemA)

        tail = lax.cond(
            even,
            lambda tl: filter_chunk(c, sbufA, dbufA, esemA, tl),
            lambda tl: filter_chunk(c, sbufB, dbufB, esemB, tl),
            tail)

        def spill_cond(th):
            return th[0] - th[1] >= SPILL

        def spill(th):
            tl, hd = th
            pltpu.sync_copy(
                ring.at[pl.ds(pl.multiple_of(hd & (CAP - 1), 8), SPILL)],
                clist.at[pl.ds(pl.multiple_of(wid * EC + hd, 8), SPILL)])
            return (tl, hd + SPILL)

        tail, head = lax.while_loop(spill_cond, spill, (tail, head))
        return (tail, head)

    tail, head = lax.fori_loop(0, nchunks, chunk_body,
                               (jnp.int32(0), jnp.int32(0)))

    iota16 = lax.iota(jnp.int32, 16)
    fill16 = jnp.full((16,), SEG << 14, jnp.int32)
    for k in range(4):
        pos = (tail + iota16 + 16 * k) & (CAP - 1)
        plsc.store_scatter(ring, [pos], fill16)

    def flush_cond(th):
        return th[1] < th[0]

    def flush(th):
        tl, hd = th
        pltpu.sync_copy(
            ring.at[pl.ds(pl.multiple_of(hd & (CAP - 1), 8), 64)],
            clist.at[pl.ds(pl.multiple_of(wid * EC + hd, 8), 64)])
        return (tl, hd + 64)

    tail, head = lax.while_loop(flush_cond, flush, (tail, head))

    cntb[...] = jnp.zeros((16,), jnp.int32) + tail
    pltpu.sync_copy(cntb, counts.at[pl.ds(pl.multiple_of(wid * 16, 8), 16)])

    bufs = ((pbufA, cidxA, clocA, rowsA, gsemA),
            (pbufB, cidxB, clocB, rowsB, gsemB))
    _apply_phase(feat, clist, acc, bufs, wid, tail)
    pltpu.sync_copy(acc.at[pl.ds(0, SEG)], aggr.at[pl.ds(lo, SEG)])


def _sc_layer2_body(feat, clist, counts, aggr,
                    pbufA, cidxA, clocA, rowsA,
                    pbufB, cidxB, clocB, rowsB,
                    cntb, acc, gsemA, gsemB):
    wid = _worker_id()
    lo = pl.multiple_of(wid * SEG, 8)
    _init_acc(acc)
    pltpu.sync_copy(counts.at[pl.ds(pl.multiple_of(wid * 16, 8), 16)], cntb)
    cnt = cntb[pl.ds(0, 16)][0]
    bufs = ((pbufA, cidxA, clocA, rowsA, gsemA),
            (pbufB, cidxB, clocB, rowsB, gsemB))
    _apply_phase(feat, clist, acc, bufs, wid, cnt)
    pltpu.sync_copy(acc.at[pl.ds(0, SEG)], aggr.at[pl.ds(lo, SEG)])


_apply_scratch = [
    pltpu.VMEM((SUP,), jnp.int32),
    pltpu.VMEM((SUP,), jnp.int32),
    pltpu.VMEM((SUP,), jnp.int32),
    pltpu.VMEM((SUP, D), jnp.float32),
    pltpu.VMEM((SUP,), jnp.int32),
    pltpu.VMEM((SUP,), jnp.int32),
    pltpu.VMEM((SUP,), jnp.int32),
    pltpu.VMEM((SUP, D), jnp.float32),
]

_sc_layer1 = pl.kernel(
    _sc_layer1_body,
    out_type=(
        jax.ShapeDtypeStruct((NPAD, D), jnp.float32),
        jax.ShapeDtypeStruct((NW * EC,), jnp.int32),
        jax.ShapeDtypeStruct((NW * 16,), jnp.int32),
    ),
    mesh=_mesh,
    compiler_params=pltpu.CompilerParams(needs_layout_passes=False),
    scratch_types=[
        pltpu.VMEM((CHUNK,), jnp.int32),
        pltpu.VMEM((CHUNK,), jnp.int32),
        pltpu.VMEM((CHUNK,), jnp.int32),
        pltpu.VMEM((CHUNK,), jnp.int32),
        pltpu.VMEM((CAP,), jnp.int32),
    ] + _apply_scratch + [
        pltpu.VMEM((16,), jnp.int32),
        pltpu.VMEM((SEG + 1, D), jnp.float32),
        pltpu.SemaphoreType.DMA,
        pltpu.SemaphoreType.DMA,
        pltpu.SemaphoreType.DMA,
        pltpu.SemaphoreType.DMA,
    ],
)

_sc_layer2 = pl.kernel(
    _sc_layer2_body,
    out_type=jax.ShapeDtypeStruct((NPAD, D), jnp.float32),
    mesh=_mesh,
    compiler_params=pltpu.CompilerParams(needs_layout_passes=False),
    scratch_types=_apply_scratch + [
        pltpu.VMEM((16,), jnp.int32),
        pltpu.VMEM((SEG + 1, D), jnp.float32),
        pltpu.SemaphoreType.DMA,
        pltpu.SemaphoreType.DMA,
    ],
)


ROW_BLK = 1000


def _dense_layer_body(x_ref, a_ref, wa_ref, wb_ref, b_ref, g_ref, bt_ref, o_ref,
                      *, extra_relu):
    x = x_ref[...]
    a = a_ref[...]
    a = jnp.where(jnp.isfinite(a), a, 0.0)
    h = (jnp.dot(x, wa_ref[...], preferred_element_type=jnp.float32)
         + jnp.dot(a, wb_ref[...], preferred_element_type=jnp.float32)
         + b_ref[...])
    h = jnp.maximum(h, 0.0)
    mean = jnp.mean(h, axis=-1, keepdims=True)
    var = jnp.mean((h - mean) * (h - mean), axis=-1, keepdims=True)
    h = (h - mean) * jax.lax.rsqrt(var + 1e-5) * g_ref[...] + bt_ref[...]
    if extra_relu:
        h = jnp.maximum(h, 0.0)
    o_ref[...] = h


def _dense_layer(x, aggr, W, b, g, bt, d_out, extra_relu):
    d_in = x.shape[1]
    n = x.shape[0]
    wa = W[:d_in]
    wb = W[d_in:]
    grid = (n // ROW_BLK,)
    return pl.pallas_call(
        functools.partial(_dense_layer_body, extra_relu=extra_relu),
        grid=grid,
        in_specs=[
            pl.BlockSpec((ROW_BLK, d_in), lambda i: (i, 0)),
            pl.BlockSpec((ROW_BLK, d_in), lambda i: (i, 0)),
            pl.BlockSpec((d_in, d_out), lambda i: (0, 0)),
            pl.BlockSpec((d_in, d_out), lambda i: (0, 0)),
            pl.BlockSpec((d_out,), lambda i: (0,)),
            pl.BlockSpec((d_out,), lambda i: (0,)),
            pl.BlockSpec((d_out,), lambda i: (0,)),
        ],
        out_specs=pl.BlockSpec((ROW_BLK, d_out), lambda i: (i, 0)),
        out_shape=jax.ShapeDtypeStruct((n, d_out), jnp.float32),
    )(x, aggr, wa, wb, b, g, bt)


def kernel(x, edge_index, W1, b1, g1, bt1, W2, b2, g2, bt2):
    src = jnp.asarray(edge_index[0], jnp.int32)
    dst = jnp.asarray(edge_index[1], jnp.int32)

    aggr1p, clist, counts = _sc_layer1(x, src, dst)
    h = _dense_layer(x, aggr1p[:N], W1, b1, g1, bt1, 128, extra_relu=True)

    aggr2p = _sc_layer2(h, clist, counts)
    out = _dense_layer(h, aggr2p[:N], W2, b2, g2, bt2, 64, extra_relu=False)
    return out
